# Initial kernel scaffold; baseline (speedup 1.0000x reference)
import jax
import jax.numpy as jnp
from jax.experimental import pallas as pl


def kernel(x_dense, x_cat, W_bot1, b_bot1, W_bot2, b_bot2, emb_tables, W_top1, b_top1, W_top2, b_top2, W_top3, b_top3):
    raise NotImplementedError("write your pallas kernel here")


# speedup vs baseline: 1.7702x; 1.7702x over previous
import functools

import numpy as np
import jax
import jax.numpy as jnp
from jax import lax
from jax.experimental import pallas as pl
from jax.experimental.pallas import tpu as pltpu
from jax.experimental.pallas import tpu_sc as plsc

B = 16384
D = 13
V = 100000
F = 26
E = 2

_NC = 2
_NS = 16
_NW = _NC * _NS
_ROWS_W = B // _NW
_IDX_W = _ROWS_W * F
_PAT = 208
_NVEC = _PAT // 16
_OFF_PAT = np.asarray([(i % F) * V for i in range(_PAT)], dtype=np.int32)


def _sc_gather(emb_flat, xcat_flat, off_pat):
    mesh = plsc.VectorSubcoreMesh(core_axis_name="c", subcore_axis_name="s")

    @functools.partial(
        pl.kernel,
        mesh=mesh,
        out_type=(jax.ShapeDtypeStruct((B * F,), jnp.float32),
                  jax.ShapeDtypeStruct((B * F,), jnp.float32)),
        scratch_types=[
            pltpu.VMEM((_IDX_W,), jnp.int32),
            pltpu.VMEM((_IDX_W,), jnp.int32),
            pltpu.VMEM((_IDX_W,), jnp.float32),
            pltpu.VMEM((_IDX_W,), jnp.float32),
            pltpu.VMEM((_PAT,), jnp.int32),
            pltpu.SemaphoreType.DMA,
        ],
    )
    def k(emb_hbm, idx_hbm, off_hbm, oute_hbm, outo_hbm,
          xc_v, idxo_v, vals_e, vals_o, off_v, sem):
        wid = lax.axis_index("s") * _NC + lax.axis_index("c")
        base = wid * _IDX_W
        pltpu.sync_copy(off_hbm, off_v)
        pltpu.sync_copy(idx_hbm.at[pl.ds(base, _IDX_W)], xc_v)

        def body(j, carry):
            jb = j * _PAT
            for r in range(_NVEC):
                s = jb + r * 16
                flat2 = (xc_v[pl.ds(s, 16)] + off_v[pl.ds(r * 16, 16)]) * E
                xc_v[pl.ds(s, 16)] = flat2
                idxo_v[pl.ds(s, 16)] = flat2 + 1
            return carry

        lax.fori_loop(0, _IDX_W // _PAT, body, 0)
        c1 = pltpu.async_copy(emb_hbm.at[xc_v], vals_e, sem)
        c2 = pltpu.async_copy(emb_hbm.at[idxo_v], vals_o, sem)
        c1.wait()
        c2.wait()
        pltpu.sync_copy(vals_e, oute_hbm.at[pl.ds(base, _IDX_W)])
        pltpu.sync_copy(vals_o, outo_hbm.at[pl.ds(base, _IDX_W)])

    return k(emb_flat, xcat_flat, off_pat)


def _mlp_body(xd_ref, embe_ref, embo_ref, w1_ref, b1_ref, w2_ref, b2_ref,
              wt1d_ref, wt1e_ref, wt1o_ref, bt1_ref, wt2_ref, bt2_ref,
              wt3_ref, bt3_ref, out_ref):
    f32 = jnp.float32
    h = jnp.dot(xd_ref[...], w1_ref[...], preferred_element_type=f32)
    h = jnp.maximum(h + b1_ref[...], 0.0)
    h = jnp.dot(h, w2_ref[...], preferred_element_type=f32)
    h = jnp.maximum(h + b2_ref[...], 0.0)
    t = (jnp.dot(h, wt1d_ref[...], preferred_element_type=f32)
         + jnp.dot(embe_ref[...], wt1e_ref[...], preferred_element_type=f32)
         + jnp.dot(embo_ref[...], wt1o_ref[...], preferred_element_type=f32)
         + bt1_ref[...])
    t = jnp.maximum(t, 0.0)
    t = jnp.dot(t, wt2_ref[...], preferred_element_type=f32)
    t = jnp.maximum(t + bt2_ref[...], 0.0)
    o = jnp.dot(t, wt3_ref[...], preferred_element_type=f32) + bt3_ref[...]
    out_ref[...] = jax.nn.sigmoid(o)


def kernel(x_dense, x_cat, W_bot1, b_bot1, W_bot2, b_bot2, emb_tables,
           W_top1, b_top1, W_top2, b_top2, W_top3, b_top3):
    emb_flat = emb_tables.reshape(-1)
    xcat_flat = x_cat.astype(jnp.int32).reshape(-1)
    off_pat = jnp.asarray(_OFF_PAT)

    emb_e, emb_o = _sc_gather(emb_flat, xcat_flat, off_pat)
    emb_e = emb_e.reshape(B, F)
    emb_o = emb_o.reshape(B, F)

    wt1 = W_top1[:, 2:].T

    BLK = 4096
    full = lambda shape: pl.BlockSpec(shape, lambda i: (0, 0))
    out = pl.pallas_call(
        _mlp_body,
        grid=(B // BLK,),
        in_specs=[
            pl.BlockSpec((BLK, D), lambda i: (i, 0)),
            pl.BlockSpec((BLK, F), lambda i: (i, 0)),
            pl.BlockSpec((BLK, F), lambda i: (i, 0)),
            full((D, 3)), full((1, 3)),
            full((3, 2)), full((1, 2)),
            full((2, 4)), full((F, 4)), full((F, 4)), full((1, 4)),
            full((4, 2)), full((1, 2)),
            full((2, 1)), full((1, 1)),
        ],
        out_specs=pl.BlockSpec((BLK, 1), lambda i: (i, 0)),
        out_shape=jax.ShapeDtypeStruct((B, 1), jnp.float32),
    )(
        x_dense, emb_e, emb_o,
        W_bot1.T, b_bot1[None, :],
        W_bot2.T, b_bot2[None, :],
        W_top1[:, :2].T, wt1[0::2], wt1[1::2], b_top1[None, :],
        W_top2.T, b_top2[None, :],
        W_top3.T, b_top3[None, :],
    )
    return out
